# baseline (speedup 1.0000x reference)
import functools
import numpy as np
import jax
import jax.numpy as jnp
from jax import lax
from jax.experimental import pallas as pl
from jax.experimental.pallas import tpu as pltpu
from jax.experimental.pallas import tpu_sc as plsc

_C = 192
_NB = 512
_NBK = 64
_OUT = 7
_NBIN = _OUT * _OUT
_TPB = _NBIN * 36
_SIZES = (128, 64, 32, 16)
_SCALES = (0.25, 0.125, 0.0625, 0.03125)
_BASES = (0, 32768, 40960, 43008)
_NROWS = 43520

_p = np.arange(_TPB) // 36
_k = np.arange(_TPB) % 36
_CF = np.stack([_p // 7, _p % 7, (_k // 4) // 3, (_k // 4) % 3]
               ).astype(np.float32)
_CI = np.stack([(_k // 4) // 3, (_k // 4) % 3, (_k % 4) // 2, (_k % 4) % 2]
               ).astype(np.int32)


def _where_chain(lvl, vals, dtype):
    r = jnp.full(lvl.shape, vals[3], dtype)
    for l in (2, 1, 0):
        r = jnp.where(lvl == l, jnp.asarray(vals[l], dtype), r)
    return r


def _prep_body(bb_ref, cf_ref, ci_ref, idx_ref, wgt_ref):
    _PH = cf_ref[0:1, :]
    _PW = cf_ref[1:2, :]
    _IYF = cf_ref[2:3, :]
    _IXF = cf_ref[3:4, :]
    _IY = ci_ref[0:1, :]
    _IX = ci_ref[1:2, :]
    _TY = ci_ref[2:3, :]
    _TX = ci_ref[3:4, :]
    bb = bb_ref[...]
    x1 = bb[:, 0:1]
    y1 = bb[:, 1:2]
    x2 = bb[:, 2:3]
    y2 = bb[:, 3:4]
    pid = pl.program_id(0)
    n = pid * _NBK + lax.broadcasted_iota(jnp.int32, (_NBK, 1), 0)
    b = (n >= 256).astype(jnp.int32)
    area = (x2 - x1) * (y2 - y1)
    t = jnp.sqrt(area) / 224.0 + 1e-8
    lvl = ((t >= 0.5).astype(jnp.int32) + (t >= 1.0).astype(jnp.int32)
           + (t >= 2.0).astype(jnp.int32))
    scale = _where_chain(lvl, _SCALES, jnp.float32)
    sf = _where_chain(lvl, [float(s) for s in _SIZES], jnp.float32)
    si = _where_chain(lvl, _SIZES, jnp.int32)
    base = _where_chain(lvl, _BASES, jnp.int32)
    hw = si * si

    x1s = x1 * scale - 0.5
    y1s = y1 * scale - 0.5
    x2s = x2 * scale - 0.5
    y2s = y2 * scale - 0.5
    roi_w = x2s - x1s
    roi_h = y2s - y1s
    bin_w = roi_w / 7.0
    bin_h = roi_h / 7.0
    gwi = jnp.clip(jnp.ceil(roi_w / 7.0), 1.0, 3.0).astype(jnp.int32)
    ghi = jnp.clip(jnp.ceil(roi_h / 7.0), 1.0, 3.0).astype(jnp.int32)
    gwf = gwi.astype(jnp.float32)
    ghf = ghi.astype(jnp.float32)
    count = gwf * ghf

    yy = y1s + _PH * bin_h + (_IYF + 0.5) * bin_h / ghf
    xx = x1s + _PW * bin_w + (_IXF + 0.5) * bin_w / gwf
    valid = (yy >= -1.0) & (yy <= sf) & (xx >= -1.0) & (xx <= sf)
    yc = jnp.clip(yy, 0.0, sf - 1.0)
    xc = jnp.clip(xx, 0.0, sf - 1.0)
    yl = jnp.minimum(jnp.floor(yc).astype(jnp.int32), si - 1)
    xl = jnp.minimum(jnp.floor(xc).astype(jnp.int32), si - 1)
    yh = jnp.minimum(yl + 1, si - 1)
    xh = jnp.minimum(xl + 1, si - 1)
    ly = yc - yl.astype(jnp.float32)
    lx = xc - xl.astype(jnp.float32)
    hy = 1.0 - ly
    hx = 1.0 - lx
    m = ((_IY < ghi) & (_IX < gwi)).astype(jnp.float32) \
        * valid.astype(jnp.float32)
    wy = jnp.where(_TY == 1, ly, hy)
    wx = jnp.where(_TX == 1, lx, hx)
    wgt_ref[...] = wy * wx * m / count
    ysel = jnp.where(_TY == 1, yh, yl)
    xsel = jnp.where(_TX == 1, xh, xl)
    idx_ref[...] = base + b * hw + ysel * si + xsel


_prep = pl.pallas_call(
    _prep_body,
    grid=(_NB // _NBK,),
    in_specs=[pl.BlockSpec((_NBK, 4), lambda i: (i, 0)),
              pl.BlockSpec((4, _TPB), lambda i: (0, 0)),
              pl.BlockSpec((4, _TPB), lambda i: (0, 0))],
    out_specs=[pl.BlockSpec((_NBK, _TPB), lambda i: (i, 0)),
               pl.BlockSpec((_NBK, _TPB), lambda i: (i, 0))],
    out_shape=[jax.ShapeDtypeStruct((_NB, _TPB), jnp.int32),
               jax.ShapeDtypeStruct((_NB, _TPB), jnp.float32)],
)

_NW = 32
_NBINS = _NB * _NBIN
_BPW = _NBINS // _NW
_KB = 2
_TPC = _KB * 36
_BB = 56
_NBLK = _BPW // _BB
_CPB = _BB // _KB
_NRING = 7
_QPB = _CPB // _NRING
_TPBK = _BB * 36


def _sc_body(idx_hbm, wgt_hbm, tab_hbm, out_hbm,
             idxv, wgtv, rows0, rows1, rows2, rows3, rows4, rows5, rows6,
             outv, siw, sg0, sg1, sg2, sg3, sg4, sg5, sg6):
    wid = lax.axis_index("s") * 2 + lax.axis_index("c")
    w0 = wid * _BPW
    ring = [(rows0, sg0), (rows1, sg1), (rows2, sg2), (rows3, sg3),
            (rows4, sg4), (rows5, sg5), (rows6, sg6)]

    def gather(c, u):
        rows, sg = ring[u]
        pltpu.async_copy(tab_hbm.at[idxv.at[pl.ds(c * _TPC, _TPC)]], rows, sg)

    def gwait(u):
        rows, sg = ring[u]
        pltpu.make_async_copy(tab_hbm.at[idxv.at[pl.ds(0, _TPC)]],
                              rows, sg).wait()

    def compute(c, u):
        rows, _ = ring[u]
        wb = c * _TPC
        wv = [wgtv[pl.ds(wb + o, 16)] for o in (0, 16, 32, 48, 56)]
        ws = [wv[4][g - 56] if g >= 56 else wv[g // 16][g % 16]
              for g in range(_TPC)]
        nj = _C // 32
        for b2 in range(_KB):
            tb = b2 * 36
            ob = _KB * c + b2
            w0v = jnp.broadcast_to(ws[tb], (16,))
            acca = [None] * nj
            accb = [None] * nj
            for j in range(nj):
                a, b = plsc.unpack(rows[tb, pl.ds(j * 32, 32)],
                                   format=plsc.PackFormat.INTERLEAVED)
                acca[j] = w0v * a
                accb[j] = w0v * b
            for k in range(1, 36):
                wv_ = jnp.broadcast_to(ws[tb + k], (16,))
                for j in range(nj):
                    a, b = plsc.unpack(rows[tb + k, pl.ds(j * 32, 32)],
                                       format=plsc.PackFormat.INTERLEAVED)
                    acca[j] = acca[j] + wv_ * a
                    accb[j] = accb[j] + wv_ * b
            for j in range(nj):
                outv[ob, pl.ds(j * 16, 16)] = acca[j]
                outv[ob, pl.ds(96 + j * 16, 16)] = accb[j]

    def block(bi, carry):
        j0 = w0 + bi * _BB
        t0 = j0 * 36
        pltpu.async_copy(idx_hbm.at[pl.ds(t0, _TPBK)], idxv, siw)
        pltpu.async_copy(wgt_hbm.at[pl.ds(t0, _TPBK)], wgtv, siw)
        pltpu.make_async_copy(idx_hbm.at[pl.ds(t0, _TPBK)], idxv, siw).wait()
        pltpu.make_async_copy(wgt_hbm.at[pl.ds(t0, _TPBK)], wgtv, siw).wait()
        for u in range(_NRING - 1):
            gather(u, u)

        def quad(q, c2):
            c0 = _NRING * q
            for u in range(_NRING):
                cu = c0 + u
                gwait(u)

                @pl.when(cu + _NRING - 1 < _CPB)
                def _():
                    gather(cu + _NRING - 1, (u + _NRING - 1) % _NRING)

                compute(cu, u)
            return c2

        lax.fori_loop(0, _QPB, quad, 0)
        pltpu.sync_copy(outv, out_hbm.at[pl.ds(j0, _BB)])
        return carry

    lax.fori_loop(0, _NBLK, block, 0)


@functools.cache
def _sc_pool():
    return functools.partial(
        pl.kernel,
        mesh=plsc.VectorSubcoreMesh(core_axis_name="c", subcore_axis_name="s"),
        compiler_params=pltpu.CompilerParams(use_tc_tiling_on_sc=False,
                                             needs_layout_passes=False),
        out_type=jax.ShapeDtypeStruct((_NBINS, _C), jnp.float32),
        scratch_types=[
            pltpu.VMEM((_TPBK,), jnp.int32),
            pltpu.VMEM((_TPBK,), jnp.float32),
        ] + [pltpu.VMEM((_TPC, _C), jnp.bfloat16)] * _NRING
          + [pltpu.VMEM((_BB, _C), jnp.float32)]
          + [pltpu.SemaphoreType.DMA] * (_NRING + 1),
    )(_sc_body)


_q = np.arange(_C)
_PERM = ((_q % 2) * 96 + 16 * (_q // 32) + (_q % 32) // 2).astype(np.int32)


def kernel(x0, x1, x2, x3, boxes0, boxes1):
    table = jnp.concatenate(
        [jnp.transpose(x, (0, 2, 3, 1)).reshape(-1, _C)
         for x in (x0, x1, x2, x3)], axis=0)
    table = table[:, _PERM].astype(jnp.bfloat16)
    bb = jnp.concatenate([boxes0, boxes1], axis=0)
    idx, wgt = _prep(bb, jnp.asarray(_CF), jnp.asarray(_CI))
    pooled = _sc_pool()(idx.reshape(-1), wgt.reshape(-1), table)
    return (pooled.reshape(_NB, _NBIN, _C)
            .transpose(0, 2, 1)
            .reshape(_NB, _C, _OUT, _OUT))


# speedup vs baseline: 1.2183x; 1.2183x over previous
import functools
import numpy as np
import jax
import jax.numpy as jnp
from jax import lax
from jax.experimental import pallas as pl
from jax.experimental.pallas import tpu as pltpu
from jax.experimental.pallas import tpu_sc as plsc

_C = 192
_NB = 512
_NBK = 64
_OUT = 7
_NBIN = _OUT * _OUT
_TPB = _NBIN * 36
_SIZES = (128, 64, 32, 16)
_SCALES = (0.25, 0.125, 0.0625, 0.03125)
_BASES = (0, 32768, 40960, 43008)
_NROWS = 43520

_p = np.arange(_TPB) // 36
_k = np.arange(_TPB) % 36
_CF = np.stack([_p // 7, _p % 7, (_k // 4) // 3, (_k // 4) % 3]
               ).astype(np.float32)
_CI = np.stack([(_k // 4) // 3, (_k // 4) % 3, (_k % 4) // 2, (_k % 4) % 2]
               ).astype(np.int32)


def _where_chain(lvl, vals, dtype):
    r = jnp.full(lvl.shape, vals[3], dtype)
    for l in (2, 1, 0):
        r = jnp.where(lvl == l, jnp.asarray(vals[l], dtype), r)
    return r


def _prep_body(bb_ref, cf_ref, ci_ref, idx_ref, wgt_ref):
    _PH = cf_ref[0:1, :]
    _PW = cf_ref[1:2, :]
    _IYF = cf_ref[2:3, :]
    _IXF = cf_ref[3:4, :]
    _IY = ci_ref[0:1, :]
    _IX = ci_ref[1:2, :]
    _TY = ci_ref[2:3, :]
    _TX = ci_ref[3:4, :]
    bb = bb_ref[...]
    x1 = bb[:, 0:1]
    y1 = bb[:, 1:2]
    x2 = bb[:, 2:3]
    y2 = bb[:, 3:4]
    pid = pl.program_id(0)
    n = pid * _NBK + lax.broadcasted_iota(jnp.int32, (_NBK, 1), 0)
    b = (n >= 256).astype(jnp.int32)
    area = (x2 - x1) * (y2 - y1)
    t = jnp.sqrt(area) / 224.0 + 1e-8
    lvl = ((t >= 0.5).astype(jnp.int32) + (t >= 1.0).astype(jnp.int32)
           + (t >= 2.0).astype(jnp.int32))
    scale = _where_chain(lvl, _SCALES, jnp.float32)
    sf = _where_chain(lvl, [float(s) for s in _SIZES], jnp.float32)
    si = _where_chain(lvl, _SIZES, jnp.int32)
    base = _where_chain(lvl, _BASES, jnp.int32)
    hw = si * si

    x1s = x1 * scale - 0.5
    y1s = y1 * scale - 0.5
    x2s = x2 * scale - 0.5
    y2s = y2 * scale - 0.5
    roi_w = x2s - x1s
    roi_h = y2s - y1s
    bin_w = roi_w / 7.0
    bin_h = roi_h / 7.0
    gwi = jnp.clip(jnp.ceil(roi_w / 7.0), 1.0, 3.0).astype(jnp.int32)
    ghi = jnp.clip(jnp.ceil(roi_h / 7.0), 1.0, 3.0).astype(jnp.int32)
    gwf = gwi.astype(jnp.float32)
    ghf = ghi.astype(jnp.float32)
    count = gwf * ghf

    yy = y1s + _PH * bin_h + (_IYF + 0.5) * bin_h / ghf
    xx = x1s + _PW * bin_w + (_IXF + 0.5) * bin_w / gwf
    valid = (yy >= -1.0) & (yy <= sf) & (xx >= -1.0) & (xx <= sf)
    yc = jnp.clip(yy, 0.0, sf - 1.0)
    xc = jnp.clip(xx, 0.0, sf - 1.0)
    yl = jnp.minimum(jnp.floor(yc).astype(jnp.int32), si - 1)
    xl = jnp.minimum(jnp.floor(xc).astype(jnp.int32), si - 1)
    yh = jnp.minimum(yl + 1, si - 1)
    xh = jnp.minimum(xl + 1, si - 1)
    ly = yc - yl.astype(jnp.float32)
    lx = xc - xl.astype(jnp.float32)
    hy = 1.0 - ly
    hx = 1.0 - lx
    m = ((_IY < ghi) & (_IX < gwi)).astype(jnp.float32) \
        * valid.astype(jnp.float32)
    wy = jnp.where(_TY == 1, ly, hy)
    wx = jnp.where(_TX == 1, lx, hx)
    wgt_ref[...] = wy * wx * m / count
    ysel = jnp.where(_TY == 1, yh, yl)
    xsel = jnp.where(_TX == 1, xh, xl)
    idx_ref[...] = base + b * hw + ysel * si + xsel


_prep = pl.pallas_call(
    _prep_body,
    grid=(_NB // _NBK,),
    in_specs=[pl.BlockSpec((_NBK, 4), lambda i: (i, 0)),
              pl.BlockSpec((4, _TPB), lambda i: (0, 0)),
              pl.BlockSpec((4, _TPB), lambda i: (0, 0))],
    out_specs=[pl.BlockSpec((_NBK, _TPB), lambda i: (i, 0)),
               pl.BlockSpec((_NBK, _TPB), lambda i: (i, 0))],
    out_shape=[jax.ShapeDtypeStruct((_NB, _TPB), jnp.int32),
               jax.ShapeDtypeStruct((_NB, _TPB), jnp.float32)],
)

_NW = 32
_NBINS = _NB * _NBIN
_BPW = _NBINS // _NW
_KB = 2
_TPC = _KB * 36
_BB = 112
_NBLK = _BPW // _BB
_CPB = _BB // _KB
_NRING = 4
_QPB = _CPB // _NRING
_TPBK = _BB * 36


def _sc_body(idx_hbm, wgt_hbm, tab_hbm, out_hbm,
             idxv, wgtv, rows0, rows1, rows2, rows3,
             outv, siw, sg0, sg1, sg2, sg3):
    wid = lax.axis_index("s") * 2 + lax.axis_index("c")
    w0 = wid * _BPW
    ring = [(rows0, sg0), (rows1, sg1), (rows2, sg2), (rows3, sg3)]

    def gather(c, u):
        rows, sg = ring[u]
        pltpu.async_copy(tab_hbm.at[idxv.at[pl.ds(c * _TPC, _TPC)]], rows, sg)

    def gwait(u):
        rows, sg = ring[u]
        pltpu.make_async_copy(tab_hbm.at[idxv.at[pl.ds(0, _TPC)]],
                              rows, sg).wait()

    def compute(c, u):
        rows, _ = ring[u]
        wb = c * _TPC
        wv = [wgtv[pl.ds(wb + o, 16)] for o in (0, 16, 32, 48, 56)]
        ws = [wv[4][g - 56] if g >= 56 else wv[g // 16][g % 16]
              for g in range(_TPC)]
        nj = _C // 32
        for b2 in range(_KB):
            tb = b2 * 36
            ob = _KB * c + b2
            w0v = jnp.broadcast_to(ws[tb], (16,))
            acca = [None] * nj
            accb = [None] * nj
            for j in range(nj):
                a, b = plsc.unpack(rows[tb, pl.ds(j * 32, 32)],
                                   format=plsc.PackFormat.INTERLEAVED)
                acca[j] = w0v * a
                accb[j] = w0v * b
            for k in range(1, 36):
                wv_ = jnp.broadcast_to(ws[tb + k], (16,))
                for j in range(nj):
                    a, b = plsc.unpack(rows[tb + k, pl.ds(j * 32, 32)],
                                       format=plsc.PackFormat.INTERLEAVED)
                    acca[j] = acca[j] + wv_ * a
                    accb[j] = accb[j] + wv_ * b
            for j in range(nj):
                outv[ob, pl.ds(j * 16, 16)] = acca[j]
                outv[ob, pl.ds(96 + j * 16, 16)] = accb[j]

    def block(bi, carry):
        j0 = w0 + bi * _BB
        t0 = j0 * 36
        pltpu.async_copy(idx_hbm.at[pl.ds(t0, _TPBK)], idxv, siw)
        pltpu.async_copy(wgt_hbm.at[pl.ds(t0, _TPBK)], wgtv, siw)
        pltpu.make_async_copy(idx_hbm.at[pl.ds(t0, _TPBK)], idxv, siw).wait()
        pltpu.make_async_copy(wgt_hbm.at[pl.ds(t0, _TPBK)], wgtv, siw).wait()
        for u in range(_NRING - 1):
            gather(u, u)

        def quad(q, c2):
            c0 = _NRING * q
            for u in range(_NRING):
                cu = c0 + u
                gwait(u)

                @pl.when(cu + _NRING - 1 < _CPB)
                def _():
                    gather(cu + _NRING - 1, (u + _NRING - 1) % _NRING)

                compute(cu, u)
            return c2

        lax.fori_loop(0, _QPB, quad, 0)
        pltpu.sync_copy(outv, out_hbm.at[pl.ds(j0, _BB)])
        return carry

    lax.fori_loop(0, _NBLK, block, 0)


@functools.cache
def _sc_pool():
    return functools.partial(
        pl.kernel,
        mesh=plsc.VectorSubcoreMesh(core_axis_name="c", subcore_axis_name="s"),
        compiler_params=pltpu.CompilerParams(use_tc_tiling_on_sc=False,
                                             needs_layout_passes=False),
        out_type=jax.ShapeDtypeStruct((_NBINS, _C), jnp.float32),
        scratch_types=[
            pltpu.VMEM((_TPBK,), jnp.int32),
            pltpu.VMEM((_TPBK,), jnp.float32),
        ] + [pltpu.VMEM((_TPC, _C), jnp.bfloat16)] * _NRING
          + [pltpu.VMEM((_BB, _C), jnp.float32)]
          + [pltpu.SemaphoreType.DMA] * (_NRING + 1),
    )(_sc_body)


_q = np.arange(_C)
_PERM = ((_q % 2) * 96 + 16 * (_q // 32) + (_q % 32) // 2).astype(np.int32)


def kernel(x0, x1, x2, x3, boxes0, boxes1):
    table = jnp.concatenate(
        [jnp.transpose(x, (0, 2, 3, 1)).reshape(-1, _C)
         for x in (x0, x1, x2, x3)], axis=0)
    table = table[:, _PERM].astype(jnp.bfloat16)
    bb = jnp.concatenate([boxes0, boxes1], axis=0)
    idx, wgt = _prep(bb, jnp.asarray(_CF), jnp.asarray(_CI))
    pooled = _sc_pool()(idx.reshape(-1), wgt.reshape(-1), table)
    return (pooled.reshape(_NB, _NBIN, _C)
            .transpose(0, 2, 1)
            .reshape(_NB, _C, _OUT, _OUT))
